# baseline (speedup 1.0000x reference)
import functools

import jax
import jax.numpy as jnp
from jax import lax
from jax.experimental import pallas as pl
from jax.experimental.pallas import tpu as pltpu
from jax.experimental.pallas import tpu_sc as plsc

LANES = 16
NC = 2
NS = 16
NW = NC * NS
CHUNK = 128


@functools.lru_cache(maxsize=None)
def _build(batch, dim):
    bpw = batch // NW
    nchunk = bpw // CHUNK
    vpr = dim // LANES
    width = 2 * dim

    mesh = plsc.VectorSubcoreMesh(core_axis_name="c", subcore_axis_name="s")

    @functools.partial(
        pl.kernel,
        out_type=jax.ShapeDtypeStruct((batch,), jnp.float32),
        mesh=mesh,
        compiler_params=pltpu.CompilerParams(use_tc_tiling_on_sc=True),
        scratch_types=[
            pltpu.VMEM((bpw,), jnp.int32),
            pltpu.VMEM((bpw,), jnp.int32),
            pltpu.VMEM((bpw,), jnp.int32),
            pltpu.VMEM((bpw,), jnp.int32),
            pltpu.VMEM((2, CHUNK, width), jnp.float32),
            pltpu.VMEM((2, CHUNK, width), jnp.float32),
            pltpu.VMEM((bpw,), jnp.float32),
            pltpu.SemaphoreType.DMA,
            pltpu.SemaphoreType.DMA,
        ],
    )
    def two_tower(uids_hbm, iids_hbm, utab_hbm, itab_hbm, out_hbm,
                  uidx_v, iidx_v, uhalf_v, ihalf_v, ubuf_v, ibuf_v, out_v,
                  sem0, sem1):
        wid = lax.axis_index("s") * NC + lax.axis_index("c")
        base = wid * bpw

        pltpu.sync_copy(uids_hbm.at[pl.ds(base, bpw)], uidx_v)
        pltpu.sync_copy(iids_hbm.at[pl.ds(base, bpw)], iidx_v)

        def halve(t, _):
            sl = pl.ds(t * LANES, LANES)
            uhalf_v[sl] = lax.shift_right_logical(uidx_v[sl], 1)
            ihalf_v[sl] = lax.shift_right_logical(iidx_v[sl], 1)
            return 0

        lax.fori_loop(0, bpw // LANES, halve, 0)

        sems = (sem0, sem1)

        def fire(j):
            b = j % 2
            idx_u = uhalf_v.at[pl.ds(j * CHUNK, CHUNK)]
            idx_i = ihalf_v.at[pl.ds(j * CHUNK, CHUNK)]
            return (
                pltpu.async_copy(utab_hbm.at[idx_u], ubuf_v.at[b], sems[b]),
                pltpu.async_copy(itab_hbm.at[idx_i], ibuf_v.at[b], sems[b]),
            )

        inflight = fire(0)
        for j in range(nchunk):
            cur = inflight
            if j + 1 < nchunk:
                inflight = fire(j + 1)
            cur[0].wait()
            cur[1].wait()

            uref = ubuf_v.at[j % 2]
            iref = ibuf_v.at[j % 2]
            lane = lax.iota(jnp.int32, LANES)

            def block(b2, _):
                row0 = b2 * LANES
                sl16 = pl.ds(j * CHUNK + row0, LANES)
                pv_u = (uidx_v[sl16] & 1).astype(jnp.float32)
                pv_i = (iidx_v[sl16] & 1).astype(jnp.float32)
                acc = jnp.zeros((LANES,), jnp.float32)
                for rr in range(LANES):
                    r = row0 + rr
                    rsplat = jnp.full((LANES,), rr, jnp.int32)
                    pu = pv_u.at[rsplat].get(mode="promise_in_bounds")
                    pi = pv_i.at[rsplat].get(mode="promise_in_bounds")
                    s = None
                    for k in range(vpr):
                        ulo = uref[r, pl.ds(k * LANES, LANES)]
                        uhi = uref[r, pl.ds(dim + k * LANES, LANES)]
                        ilo = iref[r, pl.ds(k * LANES, LANES)]
                        ihi = iref[r, pl.ds(dim + k * LANES, LANES)]
                        uu = ulo + (uhi - ulo) * pu
                        vv = ilo + (ihi - ilo) * pi
                        p = uu * vv
                        s = p if s is None else s + p
                    for h in (1, 2, 4, 8):
                        s = s + s.at[lane ^ h].get(mode="promise_in_bounds")
                    acc = jnp.where(lane == rr, s, acc)
                out_v[pl.ds(j * CHUNK + row0, LANES)] = acc
                return 0

            lax.fori_loop(0, CHUNK // LANES, block, 0)

        pltpu.sync_copy(out_v, out_hbm.at[pl.ds(base, bpw)])

    return two_tower


def kernel(user_ids, item_ids, user_table, item_table):
    batch = user_ids.shape[0]
    rows, dim = user_table.shape
    uids = jnp.asarray(user_ids, jnp.int32)
    iids = jnp.asarray(item_ids, jnp.int32)
    utab = user_table.reshape(rows // 2, 2 * dim)
    itab = item_table.reshape(rows // 2, 2 * dim)
    fn = _build(batch, dim)
    return fn(uids, iids, utab, itab)


# speedup vs baseline: 1.4921x; 1.4921x over previous
import functools

import jax
import jax.numpy as jnp
from jax import lax
from jax.experimental import pallas as pl
from jax.experimental.pallas import tpu as pltpu
from jax.experimental.pallas import tpu_sc as plsc

LANES = 16
NC = 2
NS = 16
NW = NC * NS
GRP = LANES
SUB = 8


@functools.lru_cache(maxsize=None)
def _build(batch, dim):
    bpw = batch // NW
    ngrp = bpw // GRP
    vpr = dim // LANES
    assert ngrp % 2 == 0

    mesh = plsc.VectorSubcoreMesh(core_axis_name="c", subcore_axis_name="s")

    @functools.partial(
        pl.kernel,
        out_type=jax.ShapeDtypeStruct((batch,), jnp.float32),
        mesh=mesh,
        compiler_params=pltpu.CompilerParams(use_tc_tiling_on_sc=True),
        scratch_types=[
            pltpu.VMEM((bpw,), jnp.int32),
            pltpu.VMEM((bpw,), jnp.int32),
            pltpu.VMEM((GRP * SUB, dim), jnp.float32),
            pltpu.VMEM((GRP * SUB, dim), jnp.float32),
            pltpu.VMEM((GRP * SUB, dim), jnp.float32),
            pltpu.VMEM((GRP * SUB, dim), jnp.float32),
            pltpu.VMEM((bpw,), jnp.float32),
            pltpu.SemaphoreType.DMA,
            pltpu.SemaphoreType.DMA,
        ],
    )
    def two_tower(uids_hbm, iids_hbm, utab_hbm, itab_hbm, out_hbm,
                  uidx_v, iidx_v, ubuf0, ubuf1, ibuf0, ibuf1, out_v,
                  sem0, sem1):
        wid = lax.axis_index("s") * NC + lax.axis_index("c")
        base = wid * bpw

        pltpu.sync_copy(uids_hbm.at[pl.ds(base, bpw)], uidx_v)
        pltpu.sync_copy(iids_hbm.at[pl.ds(base, bpw)], iidx_v)

        def issue(g, ubuf, ibuf, sem):
            vu = uidx_v[pl.ds(g * GRP, GRP)] & ~(SUB - 1)
            vi = iidx_v[pl.ds(g * GRP, GRP)] & ~(SUB - 1)
            for t in range(GRP):
                bu = pl.multiple_of(vu[t], SUB)
                bi = pl.multiple_of(vi[t], SUB)
                pltpu.async_copy(
                    utab_hbm.at[pl.ds(bu, SUB)],
                    ubuf.at[pl.ds(t * SUB, SUB)], sem)
                pltpu.async_copy(
                    itab_hbm.at[pl.ds(bi, SUB)],
                    ibuf.at[pl.ds(t * SUB, SUB)], sem)

        def drain(ubuf, ibuf, sem):
            pltpu.make_async_copy(
                utab_hbm.at[pl.ds(0, GRP * SUB)], ubuf, sem).wait()
            pltpu.make_async_copy(
                itab_hbm.at[pl.ds(0, GRP * SUB)], ibuf, sem).wait()

        lane = lax.iota(jnp.int32, LANES)

        def score(g, ubuf, ibuf):
            vu = uidx_v[pl.ds(g * GRP, GRP)] & (SUB - 1)
            vi = iidx_v[pl.ds(g * GRP, GRP)] & (SUB - 1)
            acc = jnp.zeros((LANES,), jnp.float32)
            for t in range(GRP):
                du = t * SUB + vu[t]
                di = t * SUB + vi[t]
                s = None
                for k in range(vpr):
                    uu = ubuf[du, pl.ds(k * LANES, LANES)]
                    vv = ibuf[di, pl.ds(k * LANES, LANES)]
                    p = uu * vv
                    s = p if s is None else s + p
                for h in (1, 2, 4, 8):
                    s = s + s.at[lane ^ h].get(mode="promise_in_bounds")
                acc = jnp.where(lane == t, s, acc)
            out_v[pl.ds(g * GRP, GRP)] = acc

        issue(0, ubuf0, ibuf0, sem0)

        def body(k, _):
            g0 = 2 * k
            issue(g0 + 1, ubuf1, ibuf1, sem1)
            drain(ubuf0, ibuf0, sem0)
            score(g0, ubuf0, ibuf0)

            @pl.when(g0 + 2 < ngrp)
            def _():
                issue(g0 + 2, ubuf0, ibuf0, sem0)

            drain(ubuf1, ibuf1, sem1)
            score(g0 + 1, ubuf1, ibuf1)
            return 0

        lax.fori_loop(0, ngrp // 2, body, 0)

        pltpu.sync_copy(out_v, out_hbm.at[pl.ds(base, bpw)])

    return two_tower


def kernel(user_ids, item_ids, user_table, item_table):
    batch = user_ids.shape[0]
    dim = user_table.shape[1]
    uids = jnp.asarray(user_ids, jnp.int32)
    iids = jnp.asarray(item_ids, jnp.int32)
    fn = _build(batch, dim)
    return fn(uids, iids, user_table, item_table)
